# baseline (speedup 1.0000x reference)
import functools

import jax
import jax.numpy as jnp
from jax import lax
from jax.experimental.layout import Format, Layout
from jax.experimental import pallas as pl
from jax.experimental.pallas import tpu as pltpu
from jax.experimental.pallas import tpu_sc as plsc

MAXD = 200
N = 100000
D = 64
NC, NS, L = 2, 16, 16
NW = NC * NS
BPW = 3200
C = 1600
NCH = BPW // C
CT = N - (NW - 1) * BPW

_mesh = plsc.VectorSubcoreMesh(core_axis_name="c", subcore_axis_name="s")


@functools.partial(
    pl.kernel,
    out_type=jax.ShapeDtypeStruct((N, D), jnp.float32),
    mesh=_mesh,
    scratch_types=[
        pltpu.VMEM((C,), jnp.int32),
        pltpu.VMEM((C,), jnp.int32),
        pltpu.VMEM((C, D), jnp.float32),
        pltpu.VMEM_SHARED((MAXD + 1, D), jnp.float32),
        pltpu.SemaphoreType.DMA,
        pltpu.SemaphoreType.DMA,
    ],
    compiler_params=pltpu.CompilerParams(use_tc_tiling_on_sc=False,
                                         needs_layout_passes=False),
)
def _encode(tt_hbm, dt_hbm, nt_hbm, dp_hbm, out_hbm,
            nt_v, d_v, rows_t, dt_sp, sem_t, sem_d):
    sid = lax.axis_index("s")
    wid = sid * NC + lax.axis_index("c")
    base_w = wid * BPW

    @pl.when(sid == 0)
    def _():
        pltpu.sync_copy(dt_hbm, dt_sp)
    plsc.subcore_barrier()

    def do_chunk(base, c, nt_vc, d_vc, rows_c):
        base = pl.multiple_of(base, 8)
        pltpu.sync_copy(nt_hbm.at[pl.ds(base, c)], nt_vc)
        pltpu.sync_copy(dp_hbm.at[pl.ds(base, c)], d_vc)
        for i in range(c // L):
            sl = pl.ds(i * L, L)
            d_vc[sl] = jnp.minimum(d_vc[sl], MAXD)
        pltpu.async_copy(tt_hbm.at[nt_vc], rows_c, sem_t).wait()
        pltpu.async_copy(dt_sp.at[d_vc], rows_c, sem_d, add=True).wait()
        pltpu.sync_copy(rows_c, out_hbm.at[pl.ds(base, c)])

    for ch in range(NCH):
        base = base_w + ch * C

        @pl.when(base + C <= N)
        def _():
            do_chunk(base, C, nt_v, d_v, rows_t)

    @pl.when(wid == NW - 1)
    def _():
        do_chunk((NW - 1) * BPW, CT,
                 nt_v.at[pl.ds(0, CT)], d_v.at[pl.ds(0, CT)],
                 rows_t.at[pl.ds(0, CT)])


def _impl(node_type, depth, type_table, depth_table):
    return _encode(type_table, depth_table,
                   node_type.astype(jnp.int32), depth.astype(jnp.int32))


_enc_jit = None


def kernel(node_type, depth, type_table, depth_table):
    global _enc_jit
    if _enc_jit is None:
        fmt = Format(Layout(major_to_minor=(0, 1), tiling=((8,),)),
                     jax.sharding.SingleDeviceSharding(jax.devices()[0]))
        _enc_jit = jax.jit(_impl, out_shardings=fmt)
    return _enc_jit(node_type, depth, type_table, depth_table)
